# Initial kernel scaffold; baseline (speedup 1.0000x reference)
import jax
import jax.numpy as jnp
from jax.experimental import pallas as pl


def kernel(x, edge_index, node_graph_ids, W1, b1, W2, b2, g_ln1, b_ln1, g_ln2, b_ln2, g_ln3, b_ln3, g_ln4, b_ln4, cW1, cb1, cW2, cb2, cW3, cb3):
    raise NotImplementedError("write your pallas kernel here")


# speedup vs baseline: 2.8673x; 2.8673x over previous
import functools

import jax
import jax.numpy as jnp
from jax import lax
from jax.experimental import pallas as pl
from jax.experimental.pallas import tpu as pltpu
from jax.experimental.pallas import tpu_sc as plsc

N = 10000
D = 128
G = 128
NC = 2
NS = 16
NW = NC * NS
N_PAD = 10240
RPT = N_PAD // NS
CHUNK = 128
CH = 80
WD = 8
E_PAD = NW * CH * CHUNK


def _sc_mesh():
    return plsc.VectorSubcoreMesh(core_axis_name="c", subcore_axis_name="s",
                                  num_cores=NC, num_subcores=NS)


def _hist(sidx, didx, ones1, zero1):
    @functools.partial(
        pl.kernel,
        out_type=jax.ShapeDtypeStruct((2, NC, N_PAD), jnp.float32),
        mesh=_sc_mesh(),
        scratch_types=[
            pltpu.VMEM_SHARED((N_PAD,), jnp.float32),
            pltpu.VMEM_SHARED((N_PAD,), jnp.float32),
            pltpu.VMEM((CHUNK,), jnp.float32),
            pltpu.VMEM((CH, CHUNK), jnp.int32),
            pltpu.VMEM((CH, CHUNK), jnp.int32),
        ],
    )
    def k(sidx_hbm, didx_hbm, ones_hbm, z_hbm, out_hbm, acc_o, acc_i, ones_v, sv, dv):
        c = lax.axis_index("c")
        s = lax.axis_index("s")
        w = s * NC + c
        row0 = s * RPT
        pltpu.sync_copy(z_hbm, acc_o.at[pl.ds(row0, RPT)])
        pltpu.sync_copy(z_hbm, acc_i.at[pl.ds(row0, RPT)])
        pltpu.sync_copy(ones_hbm, ones_v)
        pltpu.sync_copy(sidx_hbm.at[w], sv)
        pltpu.sync_copy(didx_hbm.at[w], dv)
        plsc.subcore_barrier()

        def body(j, carry):
            pltpu.sync_copy(ones_v, acc_o.at[sv.at[j]], add=True)
            pltpu.sync_copy(ones_v, acc_i.at[dv.at[j]], add=True)
            return carry

        lax.fori_loop(0, CH, body, 0)
        plsc.subcore_barrier()
        pltpu.sync_copy(acc_o.at[pl.ds(row0, RPT)], out_hbm.at[0, c, pl.ds(row0, RPT)])
        pltpu.sync_copy(acc_i.at[pl.ds(row0, RPT)], out_hbm.at[1, c, pl.ds(row0, RPT)])

    return k(sidx, didx, ones1, zero1)


def _agg(h, sidx, didx, zrows):
    @functools.partial(
        pl.kernel,
        out_type=jax.ShapeDtypeStruct((NC, N_PAD, D), jnp.float32),
        mesh=_sc_mesh(),
        scratch_types=[
            pltpu.VMEM_SHARED((N_PAD, D), jnp.float32),
            pltpu.VMEM((CH, CHUNK), jnp.int32),
            pltpu.VMEM((CH, CHUNK), jnp.int32),
            pltpu.VMEM((CHUNK, D), jnp.float32),
            pltpu.SemaphoreType.DMA,
        ],
    )
    def k(h_hbm, sidx_hbm, didx_hbm, z_hbm, out_hbm, acc, sv, dv, rows, sem):
        c = lax.axis_index("c")
        s = lax.axis_index("s")
        w = s * NC + c
        row0 = s * RPT
        pltpu.sync_copy(z_hbm, acc.at[pl.ds(row0, RPT)])
        pltpu.sync_copy(sidx_hbm.at[w], sv)
        pltpu.sync_copy(didx_hbm.at[w], dv)
        plsc.subcore_barrier()

        def body(j, carry):
            pltpu.async_copy(h_hbm.at[sv.at[j]], rows, sem).wait()
            pltpu.sync_copy(rows, acc.at[dv.at[j]], add=True)
            return carry

        lax.fori_loop(0, CH, body, 0)
        plsc.subcore_barrier()
        pltpu.sync_copy(acc.at[pl.ds(row0, RPT)], out_hbm.at[c, pl.ds(row0, RPT)])

    return k(h, sidx, didx, zrows)


def _prep(x_pad, deg4t):
    BN = 1024

    def body(x_ref, d_ref, h0_ref, ns8_ref, nd8_ref):
        d = d_ref[...]
        od = d[:, 0:1] + d[:, 1:2]
        idg = d[:, 2:3] + d[:, 3:4]
        ns = jnp.where(od > 0, 1.0 / jnp.sqrt(jnp.maximum(od, 1.0)), 0.0)
        nd = jnp.where(idg > 0, 1.0 / jnp.sqrt(jnp.maximum(idg, 1.0)), 0.0)
        ns8_ref[...] = jnp.broadcast_to(ns, ns8_ref.shape)
        nd8_ref[...] = jnp.broadcast_to(nd, nd8_ref.shape)
        h0_ref[...] = x_ref[...] * ns

    return pl.pallas_call(
        body,
        grid=(N_PAD // BN,),
        in_specs=[
            pl.BlockSpec((BN, D), lambda i: (i, 0)),
            pl.BlockSpec((BN, 4), lambda i: (i, 0)),
        ],
        out_specs=[
            pl.BlockSpec((BN, D), lambda i: (i, 0)),
            pl.BlockSpec((BN, WD), lambda i: (i, 0)),
            pl.BlockSpec((BN, WD), lambda i: (i, 0)),
        ],
        out_shape=[
            jax.ShapeDtypeStruct((N_PAD, D), jnp.float32),
            jax.ShapeDtypeStruct((N_PAD, WD), jnp.float32),
            jax.ShapeDtypeStruct((N_PAD, WD), jnp.float32),
        ],
    )(x_pad, deg4t)


def _post(parts, nd8, W, b, g_ln, b_ln, ns8):
    BN = 512
    scale = ns8 is not None

    def body(p_ref, nd_ref, W_ref, b_ref, g_ref, bl_ref, *rest):
        if scale:
            ns_ref, o_ref = rest
        else:
            (o_ref,) = rest
        p = p_ref[...]
        y = (p[0] + p[1]) * nd_ref[...][:, 0:1]
        z = jnp.dot(y, W_ref[...], preferred_element_type=jnp.float32) + b_ref[...]
        m = jnp.mean(z, axis=1, keepdims=True)
        v = jnp.mean((z - m) ** 2, axis=1, keepdims=True)
        h = jnp.maximum((z - m) / jnp.sqrt(v + 1e-5) * g_ref[...] + bl_ref[...], 0.0)
        if scale:
            h = h * ns_ref[...][:, 0:1]
        o_ref[...] = h

    in_specs = [
        pl.BlockSpec((NC, BN, D), lambda i: (0, i, 0)),
        pl.BlockSpec((BN, WD), lambda i: (i, 0)),
        pl.BlockSpec((D, D), lambda i: (0, 0)),
        pl.BlockSpec((1, D), lambda i: (0, 0)),
        pl.BlockSpec((1, D), lambda i: (0, 0)),
        pl.BlockSpec((1, D), lambda i: (0, 0)),
    ]
    args = [parts, nd8, W, b, g_ln, b_ln]
    if scale:
        in_specs.append(pl.BlockSpec((BN, WD), lambda i: (i, 0)))
        args.append(ns8)
    return pl.pallas_call(
        body,
        grid=(N_PAD // BN,),
        in_specs=in_specs,
        out_specs=pl.BlockSpec((BN, D), lambda i: (i, 0)),
        out_shape=jax.ShapeDtypeStruct((N_PAD, D), jnp.float32),
    )(*args)


_NEG = -3.0e38


def _pool(h2, ids2d):
    BN = 1024

    def body(h_ref, id_ref, cnt_ref, sum_ref, max_ref):
        i = pl.program_id(0)

        @pl.when(i == 0)
        def _():
            cnt_ref[...] = jnp.zeros_like(cnt_ref)
            sum_ref[...] = jnp.zeros_like(sum_ref)
            max_ref[...] = jnp.full_like(max_ref, _NEG)

        h = h_ref[...]
        ids = id_ref[...]
        gids = lax.broadcasted_iota(jnp.int32, (1, G), 1)
        onehot = (ids == gids).astype(jnp.float32)
        ones = jnp.ones((BN, 1), jnp.float32)
        cnt_ref[...] += lax.dot_general(
            onehot, ones, (((0,), (0,)), ((), ())), preferred_element_type=jnp.float32)
        sum_ref[...] += lax.dot_general(
            onehot, h, (((0,), (0,)), ((), ())), preferred_element_type=jnp.float32)
        rows = []
        for g in range(G):
            mg = jnp.max(jnp.where(ids == g, h, _NEG), axis=0, keepdims=True)
            rows.append(mg)
        max_ref[...] = jnp.maximum(max_ref[...], jnp.concatenate(rows, axis=0))

    return pl.pallas_call(
        body,
        grid=(N_PAD // BN,),
        in_specs=[
            pl.BlockSpec((BN, D), lambda i: (i, 0)),
            pl.BlockSpec((BN, 1), lambda i: (i, 0)),
        ],
        out_specs=[
            pl.BlockSpec((G, 1), lambda i: (0, 0)),
            pl.BlockSpec((G, D), lambda i: (0, 0)),
            pl.BlockSpec((G, D), lambda i: (0, 0)),
        ],
        out_shape=[
            jax.ShapeDtypeStruct((G, 1), jnp.float32),
            jax.ShapeDtypeStruct((G, D), jnp.float32),
            jax.ShapeDtypeStruct((G, D), jnp.float32),
        ],
    )(h2, ids2d)


def _head(cnt, hsum, hmax, cW1, cb1, g3, b3, cW2, cb2, g4, b4, cW3, cb3):
    def body(cnt_ref, sum_ref, max_ref, W1_ref, b1_ref, g3_ref, b3_ref,
             W2_ref, b2_ref, g4_ref, b4_ref, W3_ref, b3c_ref, o_ref):
        cnt_v = cnt_ref[...]
        hg_mean = sum_ref[...] / jnp.maximum(cnt_v, 1.0)
        hm = max_ref[...]
        hg_max = jnp.where(hm < -1.0e37, 0.0, hm)

        def l2(xv):
            n = jnp.sqrt(jnp.sum(xv * xv, axis=1, keepdims=True))
            return xv / jnp.maximum(n, 1e-12)

        hg = jnp.concatenate([l2(hg_mean), l2(hg_max)], axis=1)

        def lnr(z, gv, bv):
            m = jnp.mean(z, axis=1, keepdims=True)
            v = jnp.mean((z - m) ** 2, axis=1, keepdims=True)
            return jnp.maximum((z - m) / jnp.sqrt(v + 1e-5) * gv + bv, 0.0)

        o = lnr(jnp.dot(hg, W1_ref[...], preferred_element_type=jnp.float32)
                + b1_ref[...], g3_ref[...], b3_ref[...])
        o = lnr(jnp.dot(o, W2_ref[...], preferred_element_type=jnp.float32)
                + b2_ref[...], g4_ref[...], b4_ref[...])
        o_ref[...] = (jnp.dot(o, W3_ref[...], preferred_element_type=jnp.float32)
                      + b3c_ref[...])

    return pl.pallas_call(
        body,
        out_shape=jax.ShapeDtypeStruct((G, 1), jnp.float32),
    )(cnt, hsum, hmax, cW1, cb1, g3, b3, cW2, cb2, g4, b4, cW3, cb3)


def kernel(x, edge_index, node_graph_ids, W1, b1, W2, b2, g_ln1, b_ln1,
           g_ln2, b_ln2, g_ln3, b_ln3, g_ln4, b_ln4, cW1, cb1, cW2, cb2,
           cW3, cb3):
    f32 = jnp.float32
    src = edge_index[0]
    dst = edge_index[1]
    e = src.shape[0]
    dummy = jnp.full((E_PAD - e,), N, jnp.int32)
    sidx = jnp.concatenate([src, dummy]).reshape(NW, CH, CHUNK)
    didx = jnp.concatenate([dst, dummy]).reshape(NW, CH, CHUNK)
    x_pad = jnp.pad(x, ((0, N_PAD - N), (0, 0)))
    ids2d = jnp.pad(node_graph_ids, (0, N_PAD - N),
                    constant_values=-1).reshape(N_PAD, 1)
    ones1 = jnp.ones((CHUNK,), f32)
    zero1 = jnp.zeros((RPT,), f32)
    zrows = jnp.zeros((RPT, D), f32)
    r1 = lambda v: v.reshape(1, -1)

    deg = _hist(sidx, didx, ones1, zero1)
    deg4t = deg.reshape(2 * NC, N_PAD).T
    h0, ns8, nd8 = _prep(x_pad, deg4t)
    p1 = _agg(h0, sidx, didx, zrows)
    h1s = _post(p1, nd8, W1, r1(b1), r1(g_ln1), r1(b_ln1), ns8)
    p2 = _agg(h1s, sidx, didx, zrows)
    h2 = _post(p2, nd8, W2, r1(b2), r1(g_ln2), r1(b_ln2), None)
    cnt, hsum, hmax = _pool(h2, ids2d)
    return _head(cnt, hsum, hmax, cW1, r1(cb1), r1(g_ln3), r1(b_ln3),
                 cW2, r1(cb2), r1(g_ln4), r1(b_ln4), cW3, cb3.reshape(1, 1))


# speedup vs baseline: 7.2193x; 2.5178x over previous
import functools

import jax
import jax.numpy as jnp
from jax import lax
from jax.experimental import pallas as pl
from jax.experimental.pallas import tpu as pltpu
from jax.experimental.pallas import tpu_sc as plsc

N = 10000
D = 128
G = 128
NC = 2
NS = 16
NW = NC * NS
N_PAD = 10240
RPT = N_PAD // NS
CHUNK = 128
CH = 80
GC = 40
WD = 8
E_PAD = NW * CH * CHUNK


def _sc_mesh():
    return plsc.VectorSubcoreMesh(core_axis_name="c", subcore_axis_name="s",
                                  num_cores=NC, num_subcores=NS)


def _hist(sidx, didx, ones1, zero1):
    @functools.partial(
        pl.kernel,
        out_type=jax.ShapeDtypeStruct((2, NC, N_PAD), jnp.float32),
        mesh=_sc_mesh(),
        scratch_types=[
            pltpu.VMEM_SHARED((N_PAD,), jnp.float32),
            pltpu.VMEM_SHARED((N_PAD,), jnp.float32),
            pltpu.VMEM((CHUNK,), jnp.float32),
            pltpu.VMEM((CH, CHUNK), jnp.int32),
            pltpu.VMEM((CH, CHUNK), jnp.int32),
        ],
    )
    def k(sidx_hbm, didx_hbm, ones_hbm, z_hbm, out_hbm, acc_o, acc_i, ones_v, sv, dv):
        c = lax.axis_index("c")
        s = lax.axis_index("s")
        w = s * NC + c
        row0 = s * RPT
        pltpu.sync_copy(z_hbm, acc_o.at[pl.ds(row0, RPT)])
        pltpu.sync_copy(z_hbm, acc_i.at[pl.ds(row0, RPT)])
        pltpu.sync_copy(ones_hbm, ones_v)
        pltpu.sync_copy(sidx_hbm.at[w], sv)
        pltpu.sync_copy(didx_hbm.at[w], dv)
        plsc.subcore_barrier()

        def body(j, carry):
            pltpu.sync_copy(ones_v, acc_o.at[sv.at[j]], add=True)
            pltpu.sync_copy(ones_v, acc_i.at[dv.at[j]], add=True)
            return carry

        lax.fori_loop(0, CH, body, 0)
        plsc.subcore_barrier()
        pltpu.sync_copy(acc_o.at[pl.ds(row0, RPT)], out_hbm.at[0, c, pl.ds(row0, RPT)])
        pltpu.sync_copy(acc_i.at[pl.ds(row0, RPT)], out_hbm.at[1, c, pl.ds(row0, RPT)])

    return k(sidx, didx, ones1, zero1)


def _agg(h, sidx, didx, zrows):
    @functools.partial(
        pl.kernel,
        out_type=jax.ShapeDtypeStruct((NC, N_PAD, D), jnp.float32),
        mesh=_sc_mesh(),
        scratch_types=[
            pltpu.VMEM_SHARED((N_PAD, D), jnp.float32),
            pltpu.VMEM((GC, CHUNK), jnp.int32),
            pltpu.VMEM((GC, CHUNK), jnp.int32),
            pltpu.VMEM((CHUNK, D), jnp.float32),
            pltpu.VMEM((CHUNK, D), jnp.float32),
            pltpu.SemaphoreType.DMA,
            pltpu.SemaphoreType.DMA,
        ],
    )
    def k(h_hbm, sidx_hbm, didx_hbm, z_hbm, out_hbm, acc, sv, dv, rows0, rows1,
          sem0, sem1):
        c = lax.axis_index("c")
        s = lax.axis_index("s")
        w = s * NC + c
        row0 = s * RPT
        pltpu.sync_copy(z_hbm, acc.at[pl.ds(row0, RPT)])
        plsc.subcore_barrier()

        def group(g, carry):
            pltpu.sync_copy(sidx_hbm.at[w, pl.ds(g * GC, GC)], sv)
            pltpu.sync_copy(didx_hbm.at[w, pl.ds(g * GC, GC)], dv)
            pltpu.async_copy(h_hbm.at[sv.at[0]], rows0, sem0)

            def body(jj, carry2):
                j0 = jj * 2
                j1 = j0 + 1
                jn = jnp.minimum(j0 + 2, GC - 1)
                pltpu.async_copy(h_hbm.at[sv.at[j1]], rows1, sem1)
                pltpu.make_async_copy(h_hbm.at[sv.at[j0]], rows0, sem0).wait()
                pltpu.sync_copy(rows0, acc.at[dv.at[j0]], add=True)
                pltpu.async_copy(h_hbm.at[sv.at[jn]], rows0, sem0)
                pltpu.make_async_copy(h_hbm.at[sv.at[j1]], rows1, sem1).wait()
                pltpu.sync_copy(rows1, acc.at[dv.at[j1]], add=True)
                return carry2

            lax.fori_loop(0, GC // 2, body, 0)
            pltpu.make_async_copy(h_hbm.at[sv.at[0]], rows0, sem0).wait()
            return carry

        lax.fori_loop(0, CH // GC, group, 0)
        plsc.subcore_barrier()
        pltpu.sync_copy(acc.at[pl.ds(row0, RPT)], out_hbm.at[c, pl.ds(row0, RPT)])

    return k(h, sidx, didx, zrows)


def _prep(x_pad, deg4t):
    BN = 1024

    def body(x_ref, d_ref, h0_ref, ns8_ref, nd8_ref):
        d = d_ref[...]
        od = d[:, 0:1] + d[:, 1:2]
        idg = d[:, 2:3] + d[:, 3:4]
        ns = jnp.where(od > 0, 1.0 / jnp.sqrt(jnp.maximum(od, 1.0)), 0.0)
        nd = jnp.where(idg > 0, 1.0 / jnp.sqrt(jnp.maximum(idg, 1.0)), 0.0)
        ns8_ref[...] = jnp.broadcast_to(ns, ns8_ref.shape)
        nd8_ref[...] = jnp.broadcast_to(nd, nd8_ref.shape)
        h0_ref[...] = x_ref[...] * ns

    return pl.pallas_call(
        body,
        grid=(N_PAD // BN,),
        in_specs=[
            pl.BlockSpec((BN, D), lambda i: (i, 0)),
            pl.BlockSpec((BN, 4), lambda i: (i, 0)),
        ],
        out_specs=[
            pl.BlockSpec((BN, D), lambda i: (i, 0)),
            pl.BlockSpec((BN, WD), lambda i: (i, 0)),
            pl.BlockSpec((BN, WD), lambda i: (i, 0)),
        ],
        out_shape=[
            jax.ShapeDtypeStruct((N_PAD, D), jnp.float32),
            jax.ShapeDtypeStruct((N_PAD, WD), jnp.float32),
            jax.ShapeDtypeStruct((N_PAD, WD), jnp.float32),
        ],
    )(x_pad, deg4t)


def _post(parts, nd8, W, b, g_ln, b_ln, ns8):
    BN = 512
    scale = ns8 is not None

    def body(p_ref, nd_ref, W_ref, b_ref, g_ref, bl_ref, *rest):
        if scale:
            ns_ref, o_ref = rest
        else:
            (o_ref,) = rest
        p = p_ref[...]
        y = (p[0] + p[1]) * nd_ref[...][:, 0:1]
        z = jnp.dot(y, W_ref[...], preferred_element_type=jnp.float32) + b_ref[...]
        m = jnp.mean(z, axis=1, keepdims=True)
        v = jnp.mean((z - m) ** 2, axis=1, keepdims=True)
        h = jnp.maximum((z - m) / jnp.sqrt(v + 1e-5) * g_ref[...] + bl_ref[...], 0.0)
        if scale:
            h = h * ns_ref[...][:, 0:1]
        o_ref[...] = h

    in_specs = [
        pl.BlockSpec((NC, BN, D), lambda i: (0, i, 0)),
        pl.BlockSpec((BN, WD), lambda i: (i, 0)),
        pl.BlockSpec((D, D), lambda i: (0, 0)),
        pl.BlockSpec((1, D), lambda i: (0, 0)),
        pl.BlockSpec((1, D), lambda i: (0, 0)),
        pl.BlockSpec((1, D), lambda i: (0, 0)),
    ]
    args = [parts, nd8, W, b, g_ln, b_ln]
    if scale:
        in_specs.append(pl.BlockSpec((BN, WD), lambda i: (i, 0)))
        args.append(ns8)
    return pl.pallas_call(
        body,
        grid=(N_PAD // BN,),
        in_specs=in_specs,
        out_specs=pl.BlockSpec((BN, D), lambda i: (i, 0)),
        out_shape=jax.ShapeDtypeStruct((N_PAD, D), jnp.float32),
    )(*args)


_NEG = -3.0e38


def _pool(h2, ids2d):
    BN = 1024

    def body(h_ref, id_ref, cnt_ref, sum_ref, max_ref):
        i = pl.program_id(0)

        @pl.when(i == 0)
        def _():
            cnt_ref[...] = jnp.zeros_like(cnt_ref)
            sum_ref[...] = jnp.zeros_like(sum_ref)
            max_ref[...] = jnp.full_like(max_ref, _NEG)

        h = h_ref[...]
        ids = id_ref[...]
        gids = lax.broadcasted_iota(jnp.int32, (1, G), 1)
        onehot = (ids == gids).astype(jnp.float32)
        ones = jnp.ones((BN, 1), jnp.float32)
        cnt_ref[...] += lax.dot_general(
            onehot, ones, (((0,), (0,)), ((), ())), preferred_element_type=jnp.float32)
        sum_ref[...] += lax.dot_general(
            onehot, h, (((0,), (0,)), ((), ())), preferred_element_type=jnp.float32)
        rows = []
        for g in range(G):
            mg = jnp.max(jnp.where(ids == g, h, _NEG), axis=0, keepdims=True)
            rows.append(mg)
        max_ref[...] = jnp.maximum(max_ref[...], jnp.concatenate(rows, axis=0))

    return pl.pallas_call(
        body,
        grid=(N_PAD // BN,),
        in_specs=[
            pl.BlockSpec((BN, D), lambda i: (i, 0)),
            pl.BlockSpec((BN, 1), lambda i: (i, 0)),
        ],
        out_specs=[
            pl.BlockSpec((G, 1), lambda i: (0, 0)),
            pl.BlockSpec((G, D), lambda i: (0, 0)),
            pl.BlockSpec((G, D), lambda i: (0, 0)),
        ],
        out_shape=[
            jax.ShapeDtypeStruct((G, 1), jnp.float32),
            jax.ShapeDtypeStruct((G, D), jnp.float32),
            jax.ShapeDtypeStruct((G, D), jnp.float32),
        ],
    )(h2, ids2d)


def _head(cnt, hsum, hmax, cW1, cb1, g3, b3, cW2, cb2, g4, b4, cW3, cb3):
    def body(cnt_ref, sum_ref, max_ref, W1_ref, b1_ref, g3_ref, b3_ref,
             W2_ref, b2_ref, g4_ref, b4_ref, W3_ref, b3c_ref, o_ref):
        cnt_v = cnt_ref[...]
        hg_mean = sum_ref[...] / jnp.maximum(cnt_v, 1.0)
        hm = max_ref[...]
        hg_max = jnp.where(hm < -1.0e37, 0.0, hm)

        def l2(xv):
            n = jnp.sqrt(jnp.sum(xv * xv, axis=1, keepdims=True))
            return xv / jnp.maximum(n, 1e-12)

        hg = jnp.concatenate([l2(hg_mean), l2(hg_max)], axis=1)

        def lnr(z, gv, bv):
            m = jnp.mean(z, axis=1, keepdims=True)
            v = jnp.mean((z - m) ** 2, axis=1, keepdims=True)
            return jnp.maximum((z - m) / jnp.sqrt(v + 1e-5) * gv + bv, 0.0)

        o = lnr(jnp.dot(hg, W1_ref[...], preferred_element_type=jnp.float32)
                + b1_ref[...], g3_ref[...], b3_ref[...])
        o = lnr(jnp.dot(o, W2_ref[...], preferred_element_type=jnp.float32)
                + b2_ref[...], g4_ref[...], b4_ref[...])
        o_ref[...] = (jnp.dot(o, W3_ref[...], preferred_element_type=jnp.float32)
                      + b3c_ref[...])

    return pl.pallas_call(
        body,
        out_shape=jax.ShapeDtypeStruct((G, 1), jnp.float32),
    )(cnt, hsum, hmax, cW1, cb1, g3, b3, cW2, cb2, g4, b4, cW3, cb3)


def kernel(x, edge_index, node_graph_ids, W1, b1, W2, b2, g_ln1, b_ln1,
           g_ln2, b_ln2, g_ln3, b_ln3, g_ln4, b_ln4, cW1, cb1, cW2, cb2,
           cW3, cb3):
    f32 = jnp.float32
    src = edge_index[0]
    dst = edge_index[1]
    e = src.shape[0]
    dummy = N + jnp.arange(E_PAD - e, dtype=jnp.int32) % (N_PAD - N)
    sidx = jnp.concatenate([src, dummy]).reshape(NW, CH, CHUNK)
    didx = jnp.concatenate([dst, dummy]).reshape(NW, CH, CHUNK)
    x_pad = jnp.pad(x, ((0, N_PAD - N), (0, 0)))
    ids2d = jnp.pad(node_graph_ids, (0, N_PAD - N),
                    constant_values=-1).reshape(N_PAD, 1)
    ones1 = jnp.ones((CHUNK,), f32)
    zero1 = jnp.zeros((RPT,), f32)
    zrows = jnp.zeros((RPT, D), f32)
    r1 = lambda v: v.reshape(1, -1)

    deg = _hist(sidx, didx, ones1, zero1)
    deg4t = deg.reshape(2 * NC, N_PAD).T
    h0, ns8, nd8 = _prep(x_pad, deg4t)
    p1 = _agg(h0, sidx, didx, zrows)
    h1s = _post(p1, nd8, W1, r1(b1), r1(g_ln1), r1(b_ln1), ns8)
    p2 = _agg(h1s, sidx, didx, zrows)
    h2 = _post(p2, nd8, W2, r1(b2), r1(g_ln2), r1(b_ln2), None)
    cnt, hsum, hmax = _pool(h2, ids2d)
    return _head(cnt, hsum, hmax, cW1, r1(cb1), r1(g_ln3), r1(b_ln3),
                 cW2, r1(cb2), r1(g_ln4), r1(b_ln4), cW3, cb3.reshape(1, 1))


# speedup vs baseline: 10.4900x; 1.4530x over previous
import functools

import jax
import jax.numpy as jnp
from jax import lax
from jax.experimental import pallas as pl
from jax.experimental.pallas import tpu as pltpu
from jax.experimental.pallas import tpu_sc as plsc

N = 10000
D = 128
G = 128
NC = 2
NS = 16
NW = NC * NS
N_PAD = 10240
RPT = N_PAD // NS
CHUNK = 128
CH = 80
GC = 40
WD = 8
E_PAD = NW * CH * CHUNK


def _sc_mesh():
    return plsc.VectorSubcoreMesh(core_axis_name="c", subcore_axis_name="s",
                                  num_cores=NC, num_subcores=NS)


def _hist(sidx, didx, ones1, zero1):
    @functools.partial(
        pl.kernel,
        out_type=jax.ShapeDtypeStruct((2, NC, N_PAD), jnp.float32),
        mesh=_sc_mesh(),
        scratch_types=[
            pltpu.VMEM_SHARED((N_PAD,), jnp.float32),
            pltpu.VMEM_SHARED((N_PAD,), jnp.float32),
            pltpu.VMEM((CHUNK,), jnp.float32),
            pltpu.VMEM((CH, CHUNK), jnp.int32),
            pltpu.VMEM((CH, CHUNK), jnp.int32),
        ],
    )
    def k(sidx_hbm, didx_hbm, ones_hbm, z_hbm, out_hbm, acc_o, acc_i, ones_v, sv, dv):
        c = lax.axis_index("c")
        s = lax.axis_index("s")
        w = s * NC + c
        row0 = s * RPT
        pltpu.sync_copy(z_hbm, acc_o.at[pl.ds(row0, RPT)])
        pltpu.sync_copy(z_hbm, acc_i.at[pl.ds(row0, RPT)])
        pltpu.sync_copy(ones_hbm, ones_v)
        pltpu.sync_copy(sidx_hbm.at[w], sv)
        pltpu.sync_copy(didx_hbm.at[w], dv)
        plsc.subcore_barrier()

        def body(j, carry):
            pltpu.sync_copy(ones_v, acc_o.at[sv.at[j]], add=True)
            pltpu.sync_copy(ones_v, acc_i.at[dv.at[j]], add=True)
            return carry

        lax.fori_loop(0, CH, body, 0)
        plsc.subcore_barrier()
        pltpu.sync_copy(acc_o.at[pl.ds(row0, RPT)], out_hbm.at[0, c, pl.ds(row0, RPT)])
        pltpu.sync_copy(acc_i.at[pl.ds(row0, RPT)], out_hbm.at[1, c, pl.ds(row0, RPT)])

    return k(sidx, didx, ones1, zero1)


def _agg(h, sidx, didx, zrows):
    @functools.partial(
        pl.kernel,
        out_type=jax.ShapeDtypeStruct((NC, N_PAD, D), jnp.float32),
        mesh=_sc_mesh(),
        scratch_types=[
            pltpu.VMEM_SHARED((N_PAD, D), jnp.float32),
            pltpu.VMEM((GC, CHUNK), jnp.int32),
            pltpu.VMEM((GC, CHUNK), jnp.int32),
            pltpu.VMEM((CHUNK, D), jnp.float32),
            pltpu.VMEM((CHUNK, D), jnp.float32),
            pltpu.SemaphoreType.DMA,
            pltpu.SemaphoreType.DMA,
        ],
    )
    def k(h_hbm, sidx_hbm, didx_hbm, z_hbm, out_hbm, acc, sv, dv, rows0, rows1,
          sem0, sem1):
        c = lax.axis_index("c")
        s = lax.axis_index("s")
        w = s * NC + c
        row0 = s * RPT
        pltpu.sync_copy(z_hbm, acc.at[pl.ds(row0, RPT)])
        plsc.subcore_barrier()

        def group(g, carry):
            pltpu.sync_copy(sidx_hbm.at[w, pl.ds(g * GC, GC)], sv)
            pltpu.sync_copy(didx_hbm.at[w, pl.ds(g * GC, GC)], dv)
            pltpu.async_copy(h_hbm.at[sv.at[0]], rows0, sem0)

            def body(jj, carry2):
                j0 = jj * 2
                j1 = j0 + 1
                jn = jnp.minimum(j0 + 2, GC - 1)
                pltpu.async_copy(h_hbm.at[sv.at[j1]], rows1, sem1)
                pltpu.make_async_copy(h_hbm.at[sv.at[j0]], rows0, sem0).wait()
                pltpu.sync_copy(rows0, acc.at[dv.at[j0]], add=True)
                pltpu.async_copy(h_hbm.at[sv.at[jn]], rows0, sem0)
                pltpu.make_async_copy(h_hbm.at[sv.at[j1]], rows1, sem1).wait()
                pltpu.sync_copy(rows1, acc.at[dv.at[j1]], add=True)
                return carry2

            lax.fori_loop(0, GC // 2, body, 0)
            pltpu.make_async_copy(h_hbm.at[sv.at[0]], rows0, sem0).wait()
            return carry

        lax.fori_loop(0, CH // GC, group, 0)
        plsc.subcore_barrier()
        pltpu.sync_copy(acc.at[pl.ds(row0, RPT)], out_hbm.at[c, pl.ds(row0, RPT)])

    return k(h, sidx, didx, zrows)


GP = G + 8


def _scpool(h2, ids_pad, zrows):
    ROWS = N_PAD // NW

    @functools.partial(
        pl.kernel,
        out_type=jax.ShapeDtypeStruct((NW, G, D), jnp.float32),
        mesh=_sc_mesh(),
        scratch_types=[
            pltpu.VMEM((ROWS, D), jnp.float32),
            pltpu.VMEM((ROWS,), jnp.int32),
            pltpu.VMEM((GP, D), jnp.float32),
        ],
        compiler_params=pltpu.CompilerParams(needs_layout_passes=False),
    )
    def k(h_hbm, ids_hbm, z_hbm, out_hbm, hv, iv, res):
        c = lax.axis_index("c")
        s = lax.axis_index("s")
        w = s * NC + c
        base = w * ROWS
        pltpu.sync_copy(h_hbm.at[pl.ds(base, ROWS)], hv)
        pltpu.sync_copy(ids_hbm.at[pl.ds(base, ROWS)], iv)
        pltpu.sync_copy(z_hbm.at[pl.ds(0, GP)], res)
        lanes = lax.broadcasted_iota(jnp.int32, (16,), 0)

        def outer(q, carry):
            base_r = q * 16
            for j in range(16):
                r = base_r + j
                gv = plsc.load_gather(iv, [jnp.full((16,), r, jnp.int32)])
                gv = jnp.where(gv < 0, G, gv)
                for cc in range(8):
                    col = lanes + cc * 16
                    cur = plsc.load_gather(res, [gv, col])
                    vals = hv[r, pl.ds(cc * 16, 16)]
                    plsc.store_scatter(res, [gv, col], jnp.maximum(cur, vals))
            return carry

        lax.fori_loop(0, ROWS // 16, outer, 0)
        pltpu.sync_copy(res.at[pl.ds(0, G)], out_hbm.at[w])

    return k(h2, ids_pad, zrows)


def _prep(x_pad, deg4t):
    BN = 1024

    def body(x_ref, d_ref, h0_ref, ns8_ref, nd8_ref):
        d = d_ref[...]
        od = d[:, 0:1] + d[:, 1:2]
        idg = d[:, 2:3] + d[:, 3:4]
        ns = jnp.where(od > 0, 1.0 / jnp.sqrt(jnp.maximum(od, 1.0)), 0.0)
        nd = jnp.where(idg > 0, 1.0 / jnp.sqrt(jnp.maximum(idg, 1.0)), 0.0)
        ns8_ref[...] = jnp.broadcast_to(ns, ns8_ref.shape)
        nd8_ref[...] = jnp.broadcast_to(nd, nd8_ref.shape)
        h0_ref[...] = x_ref[...] * ns

    return pl.pallas_call(
        body,
        grid=(N_PAD // BN,),
        in_specs=[
            pl.BlockSpec((BN, D), lambda i: (i, 0)),
            pl.BlockSpec((BN, 4), lambda i: (i, 0)),
        ],
        out_specs=[
            pl.BlockSpec((BN, D), lambda i: (i, 0)),
            pl.BlockSpec((BN, WD), lambda i: (i, 0)),
            pl.BlockSpec((BN, WD), lambda i: (i, 0)),
        ],
        out_shape=[
            jax.ShapeDtypeStruct((N_PAD, D), jnp.float32),
            jax.ShapeDtypeStruct((N_PAD, WD), jnp.float32),
            jax.ShapeDtypeStruct((N_PAD, WD), jnp.float32),
        ],
    )(x_pad, deg4t)


def _post(parts, nd8, W, b, g_ln, b_ln, ns8):
    BN = 512
    scale = ns8 is not None

    def body(p_ref, nd_ref, W_ref, b_ref, g_ref, bl_ref, *rest):
        if scale:
            ns_ref, o_ref = rest
        else:
            (o_ref,) = rest
        p = p_ref[...]
        y = (p[0] + p[1]) * nd_ref[...][:, 0:1]
        z = jnp.dot(y, W_ref[...], preferred_element_type=jnp.float32) + b_ref[...]
        m = jnp.mean(z, axis=1, keepdims=True)
        v = jnp.mean((z - m) ** 2, axis=1, keepdims=True)
        h = jnp.maximum((z - m) / jnp.sqrt(v + 1e-5) * g_ref[...] + bl_ref[...], 0.0)
        if scale:
            h = h * ns_ref[...][:, 0:1]
        o_ref[...] = h

    in_specs = [
        pl.BlockSpec((NC, BN, D), lambda i: (0, i, 0)),
        pl.BlockSpec((BN, WD), lambda i: (i, 0)),
        pl.BlockSpec((D, D), lambda i: (0, 0)),
        pl.BlockSpec((1, D), lambda i: (0, 0)),
        pl.BlockSpec((1, D), lambda i: (0, 0)),
        pl.BlockSpec((1, D), lambda i: (0, 0)),
    ]
    args = [parts, nd8, W, b, g_ln, b_ln]
    if scale:
        in_specs.append(pl.BlockSpec((BN, WD), lambda i: (i, 0)))
        args.append(ns8)
    return pl.pallas_call(
        body,
        grid=(N_PAD // BN,),
        in_specs=in_specs,
        out_specs=pl.BlockSpec((BN, D), lambda i: (i, 0)),
        out_shape=jax.ShapeDtypeStruct((N_PAD, D), jnp.float32),
    )(*args)


def _post2pool(parts, nd8, W, b, g_ln, b_ln, ids2d):
    BN = 1024

    def body(p_ref, nd_ref, W_ref, b_ref, g_ref, bl_ref, id_ref,
             o_ref, cnt_ref, sum_ref):
        i = pl.program_id(0)

        @pl.when(i == 0)
        def _():
            cnt_ref[...] = jnp.zeros_like(cnt_ref)
            sum_ref[...] = jnp.zeros_like(sum_ref)

        p = p_ref[...]
        y = (p[0] + p[1]) * nd_ref[...][:, 0:1]
        z = jnp.dot(y, W_ref[...], preferred_element_type=jnp.float32) + b_ref[...]
        m = jnp.mean(z, axis=1, keepdims=True)
        v = jnp.mean((z - m) ** 2, axis=1, keepdims=True)
        h = jnp.maximum((z - m) / jnp.sqrt(v + 1e-5) * g_ref[...] + bl_ref[...], 0.0)
        o_ref[...] = h
        ids = id_ref[...]
        gids = lax.broadcasted_iota(jnp.int32, (1, G), 1)
        onehot = (ids == gids).astype(jnp.float32)
        ones = jnp.ones((BN, 1), jnp.float32)
        cnt_ref[...] += lax.dot_general(
            onehot, ones, (((0,), (0,)), ((), ())), preferred_element_type=jnp.float32)
        sum_ref[...] += lax.dot_general(
            onehot, h, (((0,), (0,)), ((), ())), preferred_element_type=jnp.float32)

    return pl.pallas_call(
        body,
        grid=(N_PAD // BN,),
        in_specs=[
            pl.BlockSpec((NC, BN, D), lambda i: (0, i, 0)),
            pl.BlockSpec((BN, WD), lambda i: (i, 0)),
            pl.BlockSpec((D, D), lambda i: (0, 0)),
            pl.BlockSpec((1, D), lambda i: (0, 0)),
            pl.BlockSpec((1, D), lambda i: (0, 0)),
            pl.BlockSpec((1, D), lambda i: (0, 0)),
            pl.BlockSpec((BN, 1), lambda i: (i, 0)),
        ],
        out_specs=[
            pl.BlockSpec((BN, D), lambda i: (i, 0)),
            pl.BlockSpec((G, 1), lambda i: (0, 0)),
            pl.BlockSpec((G, D), lambda i: (0, 0)),
        ],
        out_shape=[
            jax.ShapeDtypeStruct((N_PAD, D), jnp.float32),
            jax.ShapeDtypeStruct((G, 1), jnp.float32),
            jax.ShapeDtypeStruct((G, D), jnp.float32),
        ],
    )(parts, nd8, W, b, g_ln, b_ln, ids2d)


def _head(cnt, hsum, hmax, cW1, cb1, g3, b3, cW2, cb2, g4, b4, cW3, cb3):
    def body(cnt_ref, sum_ref, max_ref, W1_ref, b1_ref, g3_ref, b3_ref,
             W2_ref, b2_ref, g4_ref, b4_ref, W3_ref, b3c_ref, o_ref):
        cnt_v = cnt_ref[...]
        hg_mean = sum_ref[...] / jnp.maximum(cnt_v, 1.0)
        hg_max = jnp.max(max_ref[...], axis=0)

        def l2(xv):
            n = jnp.sqrt(jnp.sum(xv * xv, axis=1, keepdims=True))
            return xv / jnp.maximum(n, 1e-12)

        hg = jnp.concatenate([l2(hg_mean), l2(hg_max)], axis=1)

        def lnr(z, gv, bv):
            m = jnp.mean(z, axis=1, keepdims=True)
            v = jnp.mean((z - m) ** 2, axis=1, keepdims=True)
            return jnp.maximum((z - m) / jnp.sqrt(v + 1e-5) * gv + bv, 0.0)

        o = lnr(jnp.dot(hg, W1_ref[...], preferred_element_type=jnp.float32)
                + b1_ref[...], g3_ref[...], b3_ref[...])
        o = lnr(jnp.dot(o, W2_ref[...], preferred_element_type=jnp.float32)
                + b2_ref[...], g4_ref[...], b4_ref[...])
        o_ref[...] = (jnp.dot(o, W3_ref[...], preferred_element_type=jnp.float32)
                      + b3c_ref[...])

    return pl.pallas_call(
        body,
        out_shape=jax.ShapeDtypeStruct((G, 1), jnp.float32),
    )(cnt, hsum, hmax, cW1, cb1, g3, b3, cW2, cb2, g4, b4, cW3, cb3)


def kernel(x, edge_index, node_graph_ids, W1, b1, W2, b2, g_ln1, b_ln1,
           g_ln2, b_ln2, g_ln3, b_ln3, g_ln4, b_ln4, cW1, cb1, cW2, cb2,
           cW3, cb3):
    f32 = jnp.float32
    src = edge_index[0]
    dst = edge_index[1]
    e = src.shape[0]
    dummy = N + jnp.arange(E_PAD - e, dtype=jnp.int32) % (N_PAD - N)
    sidx = jnp.concatenate([src, dummy]).reshape(NW, CH, CHUNK)
    didx = jnp.concatenate([dst, dummy]).reshape(NW, CH, CHUNK)
    x_pad = jnp.pad(x, ((0, N_PAD - N), (0, 0)))
    ids2d = jnp.pad(node_graph_ids, (0, N_PAD - N),
                    constant_values=-1).reshape(N_PAD, 1)
    ones1 = jnp.ones((CHUNK,), f32)
    zero1 = jnp.zeros((RPT,), f32)
    zrows = jnp.zeros((RPT, D), f32)
    r1 = lambda v: v.reshape(1, -1)

    deg = _hist(sidx, didx, ones1, zero1)
    deg4t = deg.reshape(2 * NC, N_PAD).T
    h0, ns8, nd8 = _prep(x_pad, deg4t)
    p1 = _agg(h0, sidx, didx, zrows)
    h1s = _post(p1, nd8, W1, r1(b1), r1(g_ln1), r1(b_ln1), ns8)
    p2 = _agg(h1s, sidx, didx, zrows)
    h2, cnt, hsum = _post2pool(p2, nd8, W2, r1(b2), r1(g_ln2), r1(b_ln2), ids2d)
    hmax = _scpool(h2, ids2d.reshape(N_PAD), zrows)
    return _head(cnt, hsum, hmax, cW1, r1(cb1), r1(g_ln3), r1(b_ln3),
                 cW2, r1(cb2), r1(g_ln4), r1(b_ln4), cW3, cb3.reshape(1, 1))
